# baseline (speedup 1.0000x reference)
import functools

import jax
import jax.numpy as jnp
from jax import lax
from jax.experimental import pallas as pl
from jax.experimental.pallas import tpu as pltpu
from jax.experimental.pallas import tpu_sc as plsc


def _argmin_block_kernel(l_ref, rd_ref, idx1_ref, idx2_ref, f1t_ref, f2t_ref,
                         ckey_ref, *, n_iblocks, ti, n1, n2):
    b = pl.program_id(0)
    i = pl.program_id(1)

    L = l_ref[0]
    Rd = rd_ref[0]

    d6 = jnp.dot(L, Rd, preferred_element_type=jnp.float32)

    f1t_ref[...] = jnp.transpose(L, (1, 0))

    @pl.when(i == 0)
    def _emit_f2t():
        f2t_ref[...] = Rd * jnp.float32(-0.5)

    bits = lax.bitcast_convert_type(d6, jnp.uint32) & jnp.uint32(0xFFFFF000)
    jlane = lax.broadcasted_iota(jnp.uint32, (ti, n2), 1)
    isub = lax.broadcasted_iota(jnp.uint32, (ti, n2), 0) + jnp.uint32(i * ti)

    krow = lax.bitcast_convert_type(bits | jlane, jnp.float32)
    kcol = lax.bitcast_convert_type(bits | isub, jnp.float32)

    rk = jnp.min(krow, axis=1, keepdims=True)
    j_star = (lax.bitcast_convert_type(rk, jnp.uint32)
              & jnp.uint32(0xFFF)).astype(jnp.int32)
    idx1_ref[0] = j_star + b * n2

    ck = jnp.min(kcol, axis=0, keepdims=True)

    @pl.when(i == 0)
    def _init_cols():
        ckey_ref[...] = ck

    @pl.when(i != 0)
    def _fold_cols():
        ckey_ref[...] = jnp.minimum(ck, ckey_ref[...])

    @pl.when(i == n_iblocks - 1)
    def _emit_idx2():
        i_star = (lax.bitcast_convert_type(ckey_ref[...], jnp.uint32)
                  & jnp.uint32(0xFFF)).astype(jnp.int32)
        idx2_ref[0] = i_star + b * n1


def _make_sc_tail(n_subcores, nc, rows_per_worker, npts):
    mesh = plsc.VectorSubcoreMesh(core_axis_name="c", subcore_axis_name="s")
    groups = rows_per_worker // 16
    n_chunks = rows_per_worker // 128

    @functools.partial(
        pl.kernel, mesh=mesh,
        out_type=jax.ShapeDtypeStruct((nc * n_subcores * 32,), jnp.float32),
        scratch_types=(
            [pltpu.VMEM((128,), jnp.int32) for _ in range(6 * n_chunks)]
            + [
                pltpu.VMEM((6 * rows_per_worker,), jnp.float32),
                pltpu.VMEM((6 * rows_per_worker,), jnp.float32),
                pltpu.VMEM((32,), jnp.float32),
                pltpu.SemaphoreType.DMA,
            ]
        ),
    )
    def sc_tail(f1t, f2t, ii1, ii2, out_hbm, *scratch):
        idx_c = scratch[:6 * n_chunks]
        t_v, q_v, ostage_v, sem = scratch[6 * n_chunks:]
        c = lax.axis_index("c")
        s = lax.axis_index("s")
        wid = s * nc + c
        qbase = s * rows_per_worker

        def do_dir(fq, ft, ii):
            for k in range(n_chunks):
                pltpu.sync_copy(ii.at[pl.ds(qbase + k * 128, 128)], idx_c[k])
            for d in range(1, 6):
                for k in range(n_chunks):
                    for v in range(8):
                        sl = pl.ds(v * 16, 16)
                        idx_c[d * n_chunks + k][sl] = (
                            idx_c[k][sl] + d * npts)
            copies = []
            for d in range(6):
                for k in range(n_chunks):
                    copies.append(pltpu.async_copy(
                        ft.at[idx_c[d * n_chunks + k]],
                        t_v.at[pl.ds(d * rows_per_worker + k * 128, 128)],
                        sem))
            for d in range(6):
                pltpu.sync_copy(
                    fq.at[pl.ds(d * npts + qbase, rows_per_worker)],
                    q_v.at[pl.ds(d * rows_per_worker, rows_per_worker)])
            for cp in copies:
                cp.wait()

            acc_x = jnp.zeros((16,), jnp.float32)
            acc_n = jnp.zeros((16,), jnp.float32)
            for g in range(groups):
                q = [q_v[pl.ds(d * rows_per_worker + g * 16, 16)]
                     for d in range(6)]
                t = [t_v[pl.ds(d * rows_per_worker + g * 16, 16)]
                     for d in range(6)]
                def _sq(v):
                    return v * v
                dx = (_sq(q[0] - t[0]) + _sq(q[1] - t[1]) + _sq(q[2] - t[2]))
                dm = (_sq(q[3] - t[3]) + _sq(q[4] - t[4]) + _sq(q[5] - t[5]))
                dp = (_sq(q[3] + t[3]) + _sq(q[4] + t[4]) + _sq(q[5] + t[5]))
                acc_x = acc_x + dx
                acc_n = acc_n + jnp.minimum(dm, dp)
            ostage_v[pl.ds(0, 16)] = acc_x
            ostage_v[pl.ds(16, 16)] = acc_n
            pltpu.sync_copy(ostage_v, out_hbm.at[pl.ds(wid * 32, 32)])

        @pl.when(c == 0)
        def _dir1():
            do_dir(f1t, f2t, ii1)

        @pl.when(c != 0)
        def _dir2():
            do_dir(f2t, f1t, ii2)

    return sc_tail


def _normalize(x, eps=1e-12):
    n = jnp.sqrt(jnp.sum(x * x, axis=2, keepdims=True))
    return x / jnp.maximum(n, eps)


@jax.jit
def kernel(xyz1, xyz2, normal_rebuild, normal_gt):
    B, N1, _ = xyz1.shape
    N2 = xyz2.shape[1]

    nr = _normalize(normal_rebuild)
    ng = _normalize(normal_gt)

    sq1 = jnp.sum(xyz1 * xyz1 + nr * nr, axis=2, keepdims=True)
    sq2 = jnp.sum(xyz2 * xyz2 + ng * ng, axis=2, keepdims=True)

    ones1 = jnp.ones((B, N1, 1), jnp.float32)
    L = jnp.concatenate([xyz1, nr, sq1, ones1], axis=2)
    Rd = jnp.concatenate([-2.0 * xyz2, -2.0 * ng, ones1[:, :N2], sq2],
                         axis=2)
    Rd = jnp.transpose(Rd, (0, 2, 1))

    TI = 512 if N1 % 512 == 0 else N1
    n_iblocks = N1 // TI

    idx1, idx2, f1t, f2t = pl.pallas_call(
        functools.partial(_argmin_block_kernel, n_iblocks=n_iblocks,
                          ti=TI, n1=N1, n2=N2),
        grid=(B, n_iblocks),
        in_specs=[
            pl.BlockSpec((1, TI, 8), lambda b, i: (b, i, 0)),
            pl.BlockSpec((1, 8, N2), lambda b, i: (b, 0, 0)),
        ],
        out_specs=[
            pl.BlockSpec((1, TI, 1), lambda b, i: (b, i, 0)),
            pl.BlockSpec((1, 1, N2), lambda b, i: (b, 0, 0)),
            pl.BlockSpec((8, TI), lambda b, i: (0, b * (N1 // TI) + i)),
            pl.BlockSpec((8, N2), lambda b, i: (0, b)),
        ],
        out_shape=[
            jax.ShapeDtypeStruct((B, N1, 1), jnp.int32),
            jax.ShapeDtypeStruct((B, 1, N2), jnp.int32),
            jax.ShapeDtypeStruct((8, B * N1), jnp.float32),
            jax.ShapeDtypeStruct((8, B * N2), jnp.float32),
        ],
        scratch_shapes=[
            pltpu.VMEM((1, N2), jnp.float32),
        ],
    )(L, Rd)

    info = plsc.get_sparse_core_info()
    NC, NS = info.num_cores, info.num_subcores
    RPW = B * N1 // NS

    partials = _make_sc_tail(NS, NC, RPW, B * N1)(
        f1t.reshape(8 * B * N1), f2t.reshape(8 * B * N2),
        idx1.reshape(B * N1), idx2.reshape(B * N2)).reshape(NC * NS, 2, 16)

    inv_count = 1.0 / (B * N1)
    loss_xyz = jnp.sum(partials[:, 0, :]) * inv_count
    loss_nrm = jnp.sum(partials[:, 1, :]) * inv_count
    return (loss_xyz, loss_nrm)


# speedup vs baseline: 1.5288x; 1.5288x over previous
import functools

import jax
import jax.numpy as jnp
from jax.experimental import pallas as pl
from jax.experimental.pallas import tpu as pltpu


def _chamfer_block_kernel(l_ref, rd_ref, rn_ref, sn1_ref, oxyz_ref, onrm_ref,
                          cmin_ref, cg_ref, *, n_iblocks, inv_count):
    b = pl.program_id(0)
    i = pl.program_id(1)

    L = l_ref[0]
    Rd = rd_ref[0]
    Rn = rn_ref[0]
    sn1 = sn1_ref[0]

    d6 = jnp.dot(L, Rd, preferred_element_type=jnp.float32)
    gn = jnp.dot(L, Rn, preferred_element_type=jnp.float32)

    inf = jnp.float32(jnp.inf)

    m1 = jnp.min(d6, axis=1, keepdims=True)
    mk1 = d6 <= m1
    g1 = jnp.min(jnp.where(mk1, gn, inf), axis=1, keepdims=True)
    snn1 = sn1 + 1.0
    spx = jnp.sum(m1 - snn1 + 2.0 * g1).reshape(1, 1)
    spn = jnp.sum(snn1 - 2.0 * jnp.abs(g1)).reshape(1, 1)

    cm = jnp.min(d6, axis=0, keepdims=True)
    mk2 = d6 <= cm
    g2 = jnp.min(jnp.where(mk2, gn, inf), axis=0, keepdims=True)

    @pl.when(i == 0)
    def _init_cols():
        cmin_ref[...] = cm
        cg_ref[...] = g2

    @pl.when(i != 0)
    def _fold_cols():
        upd = cm < cmin_ref[...]
        cmin_ref[...] = jnp.where(upd, cm, cmin_ref[...])
        cg_ref[...] = jnp.where(upd, g2, cg_ref[...])

    first = jnp.logical_and(b == 0, i == 0)
    base_x = jnp.where(first, jnp.zeros((1, 1), jnp.float32), oxyz_ref[...])
    base_n = jnp.where(first, jnp.zeros((1, 1), jnp.float32), onrm_ref[...])
    acc_x = base_x + spx
    acc_n = base_n + spn
    oxyz_ref[...] = acc_x
    onrm_ref[...] = acc_n

    @pl.when(i == n_iblocks - 1)
    def _finish_batch():
        cmin = cmin_ref[...]
        cg = cg_ref[...]
        colpx = cmin - 2.0 + 2.0 * cg
        colpn = 2.0 - 2.0 * jnp.abs(cg)
        tot_x = acc_x + jnp.sum(colpx).reshape(1, 1)
        tot_n = acc_n + jnp.sum(colpn).reshape(1, 1)
        scale = jnp.where(b == pl.num_programs(0) - 1, inv_count, 1.0)
        oxyz_ref[...] = tot_x * scale
        onrm_ref[...] = tot_n * scale


def _normalize(x, eps=1e-12):
    n = jnp.sqrt(jnp.sum(x * x, axis=2, keepdims=True))
    return x / jnp.maximum(n, eps)


@jax.jit
def kernel(xyz1, xyz2, normal_rebuild, normal_gt):
    B, N1, _ = xyz1.shape
    N2 = xyz2.shape[1]

    nr = _normalize(normal_rebuild)
    ng = _normalize(normal_gt)

    sx1 = jnp.sum(xyz1 * xyz1, axis=2, keepdims=True)
    sn1 = jnp.sum(nr * nr, axis=2, keepdims=True)
    sx2 = jnp.sum(xyz2 * xyz2, axis=2, keepdims=True)
    sn2 = jnp.sum(ng * ng, axis=2, keepdims=True)

    ones1 = jnp.ones((B, N1, 1), jnp.float32)
    L = jnp.concatenate([xyz1, nr, sx1 + sn1, ones1], axis=2)
    Rd = jnp.concatenate([-2.0 * xyz2, -2.0 * ng, ones1[:, :N2],
                          sx2 + sn2], axis=2)
    Rd = jnp.transpose(Rd, (0, 2, 1))
    zeros2 = jnp.zeros((B, N2, 3), jnp.float32)
    Rn = jnp.concatenate([zeros2, ng, jnp.zeros((B, N2, 2), jnp.float32)],
                         axis=2)
    Rn = jnp.transpose(Rn, (0, 2, 1))

    TI = 1024 if N1 % 1024 == 0 else N1
    n_iblocks = N1 // TI
    inv_count = 1.0 / (B * N1)

    grid = (B, n_iblocks)
    out_xyz, out_nrm = pl.pallas_call(
        functools.partial(_chamfer_block_kernel, n_iblocks=n_iblocks,
                          inv_count=inv_count),
        grid=grid,
        in_specs=[
            pl.BlockSpec((1, TI, 8), lambda b, i: (b, i, 0)),
            pl.BlockSpec((1, 8, N2), lambda b, i: (b, 0, 0)),
            pl.BlockSpec((1, 8, N2), lambda b, i: (b, 0, 0)),
            pl.BlockSpec((1, TI, 1), lambda b, i: (b, i, 0)),
        ],
        out_specs=[
            pl.BlockSpec((1, 1), lambda b, i: (0, 0)),
            pl.BlockSpec((1, 1), lambda b, i: (0, 0)),
        ],
        out_shape=[
            jax.ShapeDtypeStruct((1, 1), jnp.float32),
            jax.ShapeDtypeStruct((1, 1), jnp.float32),
        ],
        scratch_shapes=[
            pltpu.VMEM((1, N2), jnp.float32),
            pltpu.VMEM((1, N2), jnp.float32),
        ],
    )(L, Rd, Rn, sn1)

    return (out_xyz[0, 0], out_nrm[0, 0])
